# Initial kernel scaffold; baseline (speedup 1.0000x reference)
import jax
import jax.numpy as jnp
from jax.experimental import pallas as pl


def kernel(gather, cv, VMM, w1, b1, g1, be1, w2, b2, g2, be2, w3, b3, g3, be3, wf, bf, gf, bef):
    raise NotImplementedError("write your pallas kernel here")


# speedup vs baseline: 10.7842x; 10.7842x over previous
import numpy as np
import jax
import jax.numpy as jnp
from jax import lax
from jax.experimental import pallas as pl
from jax.experimental.pallas import tpu as pltpu

_EPS = 1e-5
_T1 = 7000.0
_F32 = jnp.float32


def _lr(x):
    return jnp.where(x >= 0, x, 0.1 * x)


def _mm(a, b):
    return lax.dot_general(a, b, (((1,), (0,)), ((), ())),
                           precision=lax.Precision.HIGHEST,
                           preferred_element_type=_F32)


def _mmT(a, b):
    return lax.dot_general(a, b, (((1,), (1,)), ((), ())),
                           precision=lax.Precision.HIGHEST,
                           preferred_element_type=_F32)


def _stw(ref, a, b, c):
    row = jnp.concatenate([a.reshape(1, 1), b.reshape(1, 1), c.reshape(1, 1),
                           jnp.zeros((1, 125), _F32)], axis=1)
    ref[0] = jnp.concatenate([row, jnp.zeros((7, 128), _F32)], axis=0)


def _pack(w, kh, kw, wout, sub, cin, cout):
    rows, cols, vals = [], [], []
    for o in range(cout):
        for c in range(cin):
            for di in range(kh):
                for dj in range(kw):
                    for j in range(wout):
                        rows.append(c * (kh * sub) + sub * di + 2 * j + dj)
                        cols.append(o * wout + j)
                        vals.append(((o * cin + c) * kh + di) * kw + dj)
    p = jnp.zeros((cin * kh * sub, cout * wout), _F32)
    return p.at[np.array(rows), np.array(cols)].set(w.reshape(-1)[np.array(vals)])


def _k1(x_ref, w_ref, y_ref, st_ref):
    x = x_ref[0]
    acc = _mm(x, w_ref[...])
    y_ref[0] = acc
    _stw(st_ref, jnp.max(jnp.abs(x)), jnp.sum(acc), jnp.sum(acc * acc))


def _k2a(y1_ref, st1_ref, pv_ref, xc_ref):
    st1 = st1_ref[:, 0, :]
    s = jnp.max(st1[:, 0])
    n1 = jnp.float32(64 * 1024 * 31)
    m = jnp.sum(st1[:, 1]) / (n1 * s)
    v = jnp.sum(st1[:, 2]) / (n1 * s * s) - m * m
    a = pv_ref[0, 0] * lax.rsqrt(v + _EPS)
    z = _lr(y1_ref[0] * (a / s) + (pv_ref[0, 1] - m * a))
    ninf = jnp.full((2, 31), -jnp.inf, _F32)
    zv = jnp.concatenate([ninf, z, ninf], axis=0)
    ninh = jnp.full((1028, 2), -jnp.inf, _F32)
    zp = jnp.concatenate([ninh, zv, ninh], axis=1)
    p3 = zp[1:1025, 1:32]
    for di in range(3):
        for dj in range(3):
            p3 = jnp.maximum(p3, zp[1 + di:1025 + di, 1 + dj:32 + dj])
    p5 = zp[0:1024, 0:31]
    for di in range(5):
        for dj in range(5):
            p5 = jnp.maximum(p5, zp[di:1024 + di, dj:31 + dj])
    xc_ref[0, 0] = z
    xc_ref[0, 1] = p3
    xc_ref[0, 2] = p5


def _k2b(x_ref, w_ref, pv_ref, y2_ref, st2a_ref, st2b_ref):
    acc = jnp.zeros((341, 30), _F32)
    for c in range(3):
        acc = acc + _mm(x_ref[0, c], w_ref[c])
    outs = [acc[:, :15] + pv_ref[0, 0], acc[:, 15:] + pv_ref[0, 1]]
    y2_ref[0, 0] = outs[0]
    y2_ref[0, 1] = outs[1]
    _stw(st2a_ref, jnp.sum(outs[0]), jnp.sum(outs[0] * outs[0]),
         jnp.zeros((), _F32))
    _stw(st2b_ref, jnp.sum(outs[1]), jnp.sum(outs[1] * outs[1]),
         jnp.zeros((), _F32))


def _k3a(y2_ref, st2a_ref, st2b_ref, pv_ref, z2_ref):
    n2 = jnp.float32(64 * 341 * 15)
    st2s = [st2a_ref[:, 0, :], st2b_ref[:, 0, :]]
    for c in range(2):
        m = jnp.sum(st2s[c][:, 0]) / n2
        v = jnp.sum(st2s[c][:, 1]) / n2 - m * m
        a = pv_ref[0, c] * lax.rsqrt(v + _EPS)
        z2_ref[0, c] = _lr((y2_ref[0, c] - m) * a + pv_ref[0, 2 + c])


def _k3b(x_ref, w_ref, pv_ref, y3_ref, st3_ref):
    acc = jnp.zeros((113, 7), _F32)
    for c in range(2):
        acc = acc + _mm(x_ref[0, c], w_ref[c])
    acc = acc + pv_ref[0, 0]
    y3_ref[0] = acc
    _stw(st3_ref, jnp.sum(acc), jnp.sum(acc * acc), jnp.zeros((), _F32))


def _k4(y3_ref, st3_ref, vi_ref, r113_ref, c7_ref, rrow_ref,
        cmat_ref, wf_ref, pv_ref, yf_ref, stf_ref):
    st3 = st3_ref[:, 0, :]
    n3 = jnp.float32(64 * 113 * 7)
    m = jnp.sum(st3[:, 0]) / n3
    v = jnp.sum(st3[:, 1]) / n3 - m * m
    a3 = pv_ref[0, 0] * lax.rsqrt(v + _EPS)
    z3 = _lr((y3_ref[0] - m) * a3 + pv_ref[0, 1])
    zc = _mm(z3.reshape(8 * 113, 7), c7_ref[...]).reshape(8, 113)
    GT = _mmT(r113_ref[...], zc)
    i256 = lax.broadcasted_iota(jnp.int32, (256, 256), 1)
    rrow = rrow_ref[...]
    vi_all = vi_ref[0]
    A = jnp.zeros((1024, 256), _F32)
    for k in range(8):
        vi = vi_all[:, k:k + 1]
        maskk = jnp.where(vi == i256, jnp.float32(0.91), jnp.float32(0.01))
        A = A + GT[:, k:k + 1] * _mm(rrow, maskk)
    mix = _mm(A, cmat_ref[...])
    zv = jnp.zeros((1, 512), _F32)
    mv = jnp.concatenate([zv, mix, zv], axis=0)
    zh = jnp.zeros((1026, 1), _F32)
    mp = jnp.concatenate([zh, mv, zh], axis=1)
    acc = jnp.zeros((1024, 512), _F32)
    for di in range(3):
        for dj in range(3):
            acc = acc + wf_ref[di, dj] * mp[di:di + 1024, dj:dj + 512]
    acc = acc + pv_ref[0, 2]
    yf_ref[0] = acc
    _stw(stf_ref, jnp.sum(acc), jnp.sum(acc * acc), jnp.zeros((), _F32))


def _k5(yf_ref, stf_ref, pv_ref, o_ref):
    stf = stf_ref[:, 0, :]
    nf = jnp.float32(8 * 1024 * 512)
    m = jnp.sum(stf[:, 0]) / nf
    v = jnp.sum(stf[:, 1]) / nf - m * m
    a = pv_ref[0, 0] * lax.rsqrt(v + _EPS)
    o_ref[0] = _lr((yf_ref[0] - m) * a + pv_ref[0, 1])


def _full(shape):
    nd = len(shape)
    return pl.BlockSpec(shape, lambda m: (0,) * nd)


_PAR = pltpu.CompilerParams(dimension_semantics=("parallel",))
_S8 = pl.BlockSpec((1, 8, 128), lambda m: (m, 0, 0))
_SH8 = jax.ShapeDtypeStruct((64, 8, 128), _F32)


def kernel(gather, cv, VMM, w1, b1, g1, be1, w2, b2, g2, be2, w3, b3, g3,
           be3, wf, bf, gf, bef):
    x = gather.reshape(64, 1024, 192)
    w1p = _pack(w1, 3, 3, 31, 64, 1, 1)
    y1, st1 = pl.pallas_call(
        _k1, grid=(64,),
        in_specs=[pl.BlockSpec((1, 1024, 192), lambda m: (m, 0, 0)),
                  _full((192, 31))],
        out_specs=[pl.BlockSpec((1, 1024, 31), lambda m: (m, 0, 0)), _S8],
        out_shape=[jax.ShapeDtypeStruct((64, 1024, 31), _F32), _SH8],
        compiler_params=_PAR,
    )(x, w1p)

    pv2 = jnp.concatenate([g1, be1]).reshape(1, 2)
    xc = pl.pallas_call(
        _k2a, grid=(64,),
        in_specs=[pl.BlockSpec((1, 1024, 31), lambda m: (m, 0, 0)),
                  _full((64, 8, 128)), _full((1, 2))],
        out_specs=pl.BlockSpec((1, 3, 1024, 31), lambda m: (m, 0, 0, 0)),
        out_shape=jax.ShapeDtypeStruct((64, 3, 1024, 31), _F32),
        compiler_params=_PAR,
    )(y1, st1, pv2)

    xc2 = xc[:, :, :1023, :].reshape(64, 3, 341, 93)
    w2p = _pack(w2, 3, 3, 15, 31, 3, 2).reshape(3, 93, 30)
    pvb2 = b2.reshape(1, 2)
    y2, st2a, st2b = pl.pallas_call(
        _k2b, grid=(64,),
        in_specs=[pl.BlockSpec((1, 3, 341, 93), lambda m: (m, 0, 0, 0)),
                  _full((3, 93, 30)), _full((1, 2))],
        out_specs=[pl.BlockSpec((1, 2, 341, 15), lambda m: (m, 0, 0, 0)),
                   _S8, _S8],
        out_shape=[jax.ShapeDtypeStruct((64, 2, 341, 15), _F32), _SH8, _SH8],
        compiler_params=_PAR,
    )(xc2, w2p, pvb2)

    pv3 = jnp.concatenate([g2, be2]).reshape(1, 4)
    z2 = pl.pallas_call(
        _k3a, grid=(64,),
        in_specs=[pl.BlockSpec((1, 2, 341, 15), lambda m: (m, 0, 0, 0)),
                  _full((64, 8, 128)), _full((64, 8, 128)), _full((1, 4))],
        out_specs=pl.BlockSpec((1, 2, 341, 15), lambda m: (m, 0, 0, 0)),
        out_shape=jax.ShapeDtypeStruct((64, 2, 341, 15), _F32),
        compiler_params=_PAR,
    )(y2, st2a, st2b, pv3)

    z2r = z2[:, :, :339, :].reshape(64, 2, 113, 45)
    w3p = _pack(w3, 3, 2, 7, 15, 2, 1).reshape(2, 45, 7)
    pvb3 = b3.reshape(1, 1)
    y3, st3 = pl.pallas_call(
        _k3b, grid=(64,),
        in_specs=[pl.BlockSpec((1, 2, 113, 45), lambda m: (m, 0, 0, 0)),
                  _full((2, 45, 7)), _full((1, 1))],
        out_specs=[pl.BlockSpec((1, 113, 7), lambda m: (m, 0, 0)), _S8],
        out_shape=[jax.ShapeDtypeStruct((64, 113, 7), _F32), _SH8],
        compiler_params=_PAR,
    )(z2r, w3p, pvb3)

    i256 = jnp.eye(256, dtype=_F32)
    rrow = jax.image.resize(i256, (1024, 256), "linear")
    cmat = jax.image.resize(i256, (256, 512), "linear")
    r113 = jax.image.resize(jnp.eye(113, dtype=_F32), (1024, 113), "linear")
    c7 = jax.image.resize(jnp.eye(7, dtype=_F32), (7, 1), "linear")

    dt = (_T1 - 0.0) / 255
    t_flat = (cv[..., 0] / dt).reshape(-1, 16)
    dv = (VMM[:, 1] - VMM[:, 0]) / 255
    v_flat = ((cv[..., 1] - VMM[:, 0, None, None]) / dv[:, None, None]
              ).reshape(-1, 16)
    tgrid = jnp.arange(256, dtype=_F32)
    v_curve = jax.vmap(lambda xp, fp: jnp.interp(tgrid, xp, fp))(t_flat, v_flat)
    vi2 = jnp.clip(v_curve.astype(jnp.int32), 0, 255
                   ).reshape(8, 8, 256).transpose(0, 2, 1)
    pv4 = jnp.concatenate([g3, be3, bf]).reshape(1, 3)
    yf, stf = pl.pallas_call(
        _k4, grid=(8,),
        in_specs=[pl.BlockSpec((1, 8, 113, 7), lambda b: (b, 0, 0, 0)),
                  _full((64, 8, 128)),
                  pl.BlockSpec((1, 256, 8), lambda b: (b, 0, 0)),
                  _full((1024, 113)), _full((7, 1)), _full((1024, 256)),
                  _full((256, 512)), _full((3, 3)), _full((1, 3))],
        out_specs=[pl.BlockSpec((1, 1024, 512), lambda b: (b, 0, 0)),
                   pl.BlockSpec((1, 8, 128), lambda b: (b, 0, 0))],
        out_shape=[jax.ShapeDtypeStruct((8, 1024, 512), _F32),
                   jax.ShapeDtypeStruct((8, 8, 128), _F32)],
        compiler_params=_PAR,
    )(y3.reshape(8, 8, 113, 7), st3, vi2, r113, c7, rrow, cmat,
      wf.reshape(3, 3), pv4)

    pv5 = jnp.concatenate([gf, bef]).reshape(1, 2)
    out = pl.pallas_call(
        _k5, grid=(8,),
        in_specs=[pl.BlockSpec((1, 1024, 512), lambda b: (b, 0, 0)),
                  _full((8, 8, 128)), _full((1, 2))],
        out_specs=pl.BlockSpec((1, 1024, 512), lambda b: (b, 0, 0)),
        out_shape=jax.ShapeDtypeStruct((8, 1024, 512), _F32),
        compiler_params=_PAR,
    )(yf, stf, pv5)
    return out.reshape(8, 1, 1024, 512)
